# Initial kernel scaffold; baseline (speedup 1.0000x reference)
import functools

import jax
import jax.numpy as jnp
from jax import lax
from jax.experimental import pallas as pl
from jax.experimental.pallas import tpu as pltpu
from jax.experimental.pallas import tpu_sc as plsc

N = 10000
E = 320000
D = 128
H = 128
C = 40
EPS = 1e-5

NC = 2
NS = 16
NW = NC * NS
N_P = 10240
ROWS_PER_TILE = N_P // NS
CHUNK = 80
NCHUNK = 128
E_TILE = CHUNK * NCHUNK
E_PAD = E_TILE * NW
C_P = 64

_MESH = plsc.VectorSubcoreMesh(core_axis_name="c", subcore_axis_name="s")


@functools.partial(
    pl.kernel,
    out_type=jax.ShapeDtypeStruct((NC, N_P), jnp.float32),
    mesh=_MESH,
    scratch_types=[
        pltpu.VMEM((NCHUNK, CHUNK), jnp.int32),
        pltpu.VMEM((CHUNK,), jnp.float32),
        pltpu.VMEM_SHARED((N_P,), jnp.float32),
    ],
)
def _sc_degree(dst3_hbm, zeros1_hbm, out_hbm, dst_idx, ones_v, deg_sh):
    c = lax.axis_index("c")
    s = lax.axis_index("s")
    w = c * NS + s
    pltpu.sync_copy(dst3_hbm.at[w], dst_idx)
    for i in range(CHUNK // 16):
        ones_v[pl.ds(i * 16, 16)] = jnp.ones((16,), jnp.float32)
    rbase = s * ROWS_PER_TILE
    pltpu.sync_copy(zeros1_hbm, deg_sh.at[pl.ds(rbase, ROWS_PER_TILE)])
    plsc.subcore_barrier()

    def body(j, carry):
        pltpu.sync_copy(ones_v, deg_sh.at[dst_idx.at[j]], add=True)
        return carry

    lax.fori_loop(0, NCHUNK, body, 0)
    plsc.subcore_barrier()
    pltpu.sync_copy(deg_sh.at[pl.ds(rbase, ROWS_PER_TILE)],
                    out_hbm.at[c, pl.ds(rbase, ROWS_PER_TILE)])


def _make_sc_aggregate(hd):
    @functools.partial(
        pl.kernel,
        out_type=jax.ShapeDtypeStruct((NC, N_P, hd), jnp.float32),
        mesh=_MESH,
        scratch_types=[
            pltpu.VMEM((NCHUNK, CHUNK), jnp.int32),
            pltpu.VMEM((NCHUNK, CHUNK), jnp.int32),
            pltpu.VMEM((CHUNK, hd), jnp.float32),
            pltpu.VMEM((CHUNK, hd), jnp.float32),
            pltpu.VMEM_SHARED((N_P, hd), jnp.float32),
            pltpu.SemaphoreType.DMA,
            pltpu.SemaphoreType.DMA,
        ],
    )
    def k(hs_hbm, src3_hbm, dst3_hbm, zeros_hbm, out_hbm,
          src_idx, dst_idx, rows0, rows1, acc, sem0, sem1):
        c = lax.axis_index("c")
        s = lax.axis_index("s")
        w = c * NS + s
        pltpu.sync_copy(src3_hbm.at[w], src_idx)
        pltpu.sync_copy(dst3_hbm.at[w], dst_idx)

        rbase = s * ROWS_PER_TILE

        @pl.when(c == 0)
        def _():
            pltpu.sync_copy(hs_hbm.at[pl.ds(rbase, ROWS_PER_TILE)],
                            acc.at[pl.ds(rbase, ROWS_PER_TILE)])

        @pl.when(c != 0)
        def _():
            pltpu.sync_copy(zeros_hbm, acc.at[pl.ds(rbase, ROWS_PER_TILE)])

        plsc.subcore_barrier()

        pltpu.async_copy(hs_hbm.at[src_idx.at[0]], rows0, sem0)
        pltpu.async_copy(hs_hbm.at[src_idx.at[1]], rows1, sem1)

        def body(kk, carry):
            j0 = 2 * kk
            j1 = j0 + 1
            pltpu.make_async_copy(hs_hbm.at[pl.ds(0, CHUNK)], rows0, sem0).wait()
            pltpu.sync_copy(rows0, acc.at[dst_idx.at[j0]], add=True)

            @pl.when(kk < NCHUNK // 2 - 1)
            def _():
                pltpu.async_copy(hs_hbm.at[src_idx.at[j0 + 2]], rows0, sem0)

            pltpu.make_async_copy(hs_hbm.at[pl.ds(0, CHUNK)], rows1, sem1).wait()
            pltpu.sync_copy(rows1, acc.at[dst_idx.at[j1]], add=True)

            @pl.when(kk < NCHUNK // 2 - 1)
            def _():
                pltpu.async_copy(hs_hbm.at[src_idx.at[j1 + 2]], rows1, sem1)

            return carry

        lax.fori_loop(0, NCHUNK // 2, body, 0)
        plsc.subcore_barrier()
        pltpu.sync_copy(acc.at[pl.ds(rbase, ROWS_PER_TILE)],
                        out_hbm.at[c, pl.ds(rbase, ROWS_PER_TILE)])

    return k


_sc_aggregate_h = _make_sc_aggregate(H)
_sc_aggregate_c = _make_sc_aggregate(C_P)


def _tc_first(deg2_ref, x_ref, w1_ref, hs_ref, dinv_ref):
    deg = deg2_ref[0, :] + deg2_ref[1, :] + 1.0
    dinv = lax.rsqrt(deg)
    dinv2 = dinv[:, None]
    dinv_ref[...] = dinv2
    h = jnp.dot(x_ref[...], w1_ref[...], preferred_element_type=jnp.float32)
    hs_ref[0:N, :] = h * dinv2[0:N]
    hs_ref[N:N_P, :] = jnp.zeros((N_P - N, H), jnp.float32)


def _make_tc_mid(hd_out):
    def body(acc_ref, dinv_ref, b_ref, g_ref, be_ref, w_ref, hs_ref):
        a = acc_ref[0] + acc_ref[1]
        dinv2 = dinv_ref[0:N]
        pre = a[0:N] * dinv2 + b_ref[...]
        mu = jnp.mean(pre, axis=0)
        zc = pre - mu
        var = jnp.mean(zc * zc, axis=0)
        y = g_ref[...] * zc * lax.rsqrt(var + EPS) + be_ref[...]
        r = jnp.maximum(y, 0.0)
        h = jnp.dot(r, w_ref[...], preferred_element_type=jnp.float32)
        hs_ref[0:N, :] = h * dinv2
        hs_ref[N:N_P, :] = jnp.zeros((N_P - N, hd_out), jnp.float32)

    return body


def _tc_final(acc_ref, dinv_ref, b3_ref, out_ref):
    a = acc_ref[0] + acc_ref[1]
    pre = a[0:N, 0:C] * dinv_ref[0:N] + b3_ref[...]
    m = jnp.max(pre, axis=1, keepdims=True)
    z = pre - m
    lse = jnp.log(jnp.sum(jnp.exp(z), axis=1, keepdims=True))
    out_ref[...] = z - lse


def kernel(x, adj_t, W1, b1, g1, be1, W2, b2, g2, be2, W3, b3):
    src = adj_t[0]
    dst = adj_t[1]
    pad = jnp.full((E_PAD - E,), N_P - 1, jnp.int32)
    src3 = jnp.concatenate([src, pad]).reshape(NW, NCHUNK, CHUNK)
    dst3 = jnp.concatenate([dst, pad]).reshape(NW, NCHUNK, CHUNK)
    zeros1 = jnp.zeros((ROWS_PER_TILE,), jnp.float32)
    zeros_h = jnp.zeros((ROWS_PER_TILE, H), jnp.float32)
    zeros_c = jnp.zeros((ROWS_PER_TILE, C_P), jnp.float32)
    W3p = jnp.pad(W3, ((0, 0), (0, C_P - C)))

    deg2 = _sc_degree(dst3, zeros1)

    hs1, dinv = pl.pallas_call(
        _tc_first,
        out_shape=(jax.ShapeDtypeStruct((N_P, H), jnp.float32),
                   jax.ShapeDtypeStruct((N_P, 1), jnp.float32)),
    )(deg2, x, W1)

    acc1 = _sc_aggregate_h(hs1, src3, dst3, zeros_h)
    hs2 = pl.pallas_call(
        _make_tc_mid(H),
        out_shape=jax.ShapeDtypeStruct((N_P, H), jnp.float32),
    )(acc1, dinv, b1, g1, be1, W2)

    acc2 = _sc_aggregate_h(hs2, src3, dst3, zeros_h)
    hs3 = pl.pallas_call(
        _make_tc_mid(C_P),
        out_shape=jax.ShapeDtypeStruct((N_P, C_P), jnp.float32),
    )(acc2, dinv, b2, g2, be2, W3p)

    acc3 = _sc_aggregate_c(hs3, src3, dst3, zeros_c)
    out = pl.pallas_call(
        _tc_final,
        out_shape=jax.ShapeDtypeStruct((N, C), jnp.float32),
    )(acc3, dinv, b3)
    return out


# speedup vs baseline: 7.0299x; 7.0299x over previous
import functools

import jax
import jax.numpy as jnp
from jax import lax
from jax.experimental import pallas as pl
from jax.experimental.pallas import tpu as pltpu
from jax.experimental.pallas import tpu_sc as plsc

N = 10000
E = 320000
D = 128
H = 128
C = 40
EPS = 1e-5

NC = 2
NS = 16
NW = NC * NS
N_P = 10240
ROWS_PER_TILE = N_P // NS
CHUNK = 80
NCHUNK = 128
E_PAD = CHUNK * NCHUNK * NW

_MESH = plsc.VectorSubcoreMesh(core_axis_name="c", subcore_axis_name="s")


@functools.partial(
    pl.kernel,
    out_type=jax.ShapeDtypeStruct((NC, N_P), jnp.float32),
    mesh=_MESH,
    scratch_types=[
        pltpu.VMEM((NCHUNK, CHUNK), jnp.int32),
        pltpu.VMEM((CHUNK,), jnp.float32),
        pltpu.VMEM_SHARED((N_P,), jnp.float32),
    ],
)
def _sc_degree(dst3_hbm, zeros1_hbm, out_hbm, dst_idx, ones_v, deg_sh):
    c = lax.axis_index("c")
    s = lax.axis_index("s")
    w = c * NS + s
    pltpu.sync_copy(dst3_hbm.at[w], dst_idx)
    for i in range(CHUNK // 16):
        ones_v[pl.ds(i * 16, 16)] = jnp.ones((16,), jnp.float32)
    rbase = s * ROWS_PER_TILE
    pltpu.sync_copy(zeros1_hbm, deg_sh.at[pl.ds(rbase, ROWS_PER_TILE)])
    plsc.subcore_barrier()

    def body(j, carry):
        pltpu.sync_copy(ones_v, deg_sh.at[dst_idx.at[j]], add=True)
        return carry

    lax.fori_loop(0, NCHUNK, body, 0)
    plsc.subcore_barrier()
    pltpu.sync_copy(deg_sh.at[pl.ds(rbase, ROWS_PER_TILE)],
                    out_hbm.at[c, pl.ds(rbase, ROWS_PER_TILE)])


@functools.partial(
    pl.kernel,
    out_type=jax.ShapeDtypeStruct((NC, N_P, H), jnp.float32),
    mesh=_MESH,
    scratch_types=[
        pltpu.VMEM((CHUNK,), jnp.int32),
        pltpu.VMEM((CHUNK,), jnp.int32),
        pltpu.VMEM((CHUNK,), jnp.int32),
        pltpu.VMEM((CHUNK,), jnp.int32),
        pltpu.VMEM((CHUNK, H), jnp.float32),
        pltpu.VMEM((CHUNK, H), jnp.float32),
        pltpu.VMEM_SHARED((N_P, H), jnp.float32),
        pltpu.SemaphoreType.DMA,
        pltpu.SemaphoreType.DMA,
        pltpu.SemaphoreType.DMA,
        pltpu.SemaphoreType.DMA,
        pltpu.SemaphoreType.DMA,
        pltpu.SemaphoreType.DMA,
    ],
)
def _sc_aggregate(hs_hbm, src3_hbm, dst3_hbm, zeros_hbm, out_hbm,
                  sidx0, sidx1, didx0, didx1, rows0, rows1, acc,
                  sem_si0, sem_si1, sem_di0, sem_di1, sem_g0, sem_g1):
    c = lax.axis_index("c")
    s = lax.axis_index("s")
    w = c * NS + s
    rbase = s * ROWS_PER_TILE

    @pl.when(c == 0)
    def _():
        pltpu.sync_copy(hs_hbm.at[pl.ds(rbase, ROWS_PER_TILE)],
                        acc.at[pl.ds(rbase, ROWS_PER_TILE)])

    @pl.when(c != 0)
    def _():
        pltpu.sync_copy(zeros_hbm, acc.at[pl.ds(rbase, ROWS_PER_TILE)])

    pltpu.async_copy(src3_hbm.at[w, 0], sidx0, sem_si0)
    pltpu.async_copy(dst3_hbm.at[w, 0], didx0, sem_di0)
    pltpu.async_copy(src3_hbm.at[w, 1], sidx1, sem_si1)
    pltpu.async_copy(dst3_hbm.at[w, 1], didx1, sem_di1)

    plsc.subcore_barrier()

    pltpu.make_async_copy(src3_hbm.at[w, 0], sidx0, sem_si0).wait()
    pltpu.async_copy(hs_hbm.at[sidx0], rows0, sem_g0)
    pltpu.make_async_copy(src3_hbm.at[w, 1], sidx1, sem_si1).wait()
    pltpu.async_copy(hs_hbm.at[sidx1], rows1, sem_g1)

    def body(kk, carry):
        j0 = 2 * kk
        j1 = j0 + 1
        not_last = kk < NCHUNK // 2 - 1

        pltpu.make_async_copy(hs_hbm.at[pl.ds(0, CHUNK)], rows0, sem_g0).wait()
        pltpu.make_async_copy(dst3_hbm.at[w, j0], didx0, sem_di0).wait()
        pltpu.sync_copy(rows0, acc.at[didx0], add=True)

        @pl.when(not_last)
        def _():
            pltpu.async_copy(src3_hbm.at[w, j0 + 2], sidx0, sem_si0)
            pltpu.async_copy(dst3_hbm.at[w, j0 + 2], didx0, sem_di0)

        pltpu.make_async_copy(hs_hbm.at[pl.ds(0, CHUNK)], rows1, sem_g1).wait()
        pltpu.make_async_copy(dst3_hbm.at[w, j1], didx1, sem_di1).wait()
        pltpu.sync_copy(rows1, acc.at[didx1], add=True)

        @pl.when(not_last)
        def _():
            pltpu.async_copy(src3_hbm.at[w, j1 + 2], sidx1, sem_si1)
            pltpu.async_copy(dst3_hbm.at[w, j1 + 2], didx1, sem_di1)
            pltpu.make_async_copy(src3_hbm.at[w, 0], sidx0, sem_si0).wait()
            pltpu.async_copy(hs_hbm.at[sidx0], rows0, sem_g0)
            pltpu.make_async_copy(src3_hbm.at[w, 0], sidx1, sem_si1).wait()
            pltpu.async_copy(hs_hbm.at[sidx1], rows1, sem_g1)

        return carry

    lax.fori_loop(0, NCHUNK // 2, body, 0)
    plsc.subcore_barrier()
    pltpu.sync_copy(acc.at[pl.ds(rbase, ROWS_PER_TILE)],
                    out_hbm.at[c, pl.ds(rbase, ROWS_PER_TILE)])


def _tc_first(deg2_ref, x_ref, w1_ref, hs_ref, dinv_ref):
    deg = deg2_ref[0, :] + deg2_ref[1, :] + 1.0
    dinv = lax.rsqrt(deg)
    dinv2 = dinv[:, None]
    dinv_ref[...] = dinv2
    h = jnp.dot(x_ref[...], w1_ref[...], preferred_element_type=jnp.float32)
    hs_ref[0:N, :] = h * dinv2[0:N]
    hs_ref[N:N_P, :] = jnp.zeros((N_P - N, H), jnp.float32)


def _tc_mid(acc_ref, dinv_ref, b_ref, g_ref, be_ref, w_ref, hs_ref):
    a = acc_ref[0] + acc_ref[1]
    dinv2 = dinv_ref[0:N]
    pre = a[0:N] * dinv2 + b_ref[...]
    mu = jnp.mean(pre, axis=0)
    zc = pre - mu
    var = jnp.mean(zc * zc, axis=0)
    y = g_ref[...] * zc * lax.rsqrt(var + EPS) + be_ref[...]
    r = jnp.maximum(y, 0.0)
    h = jnp.dot(r, w_ref[...], preferred_element_type=jnp.float32)
    hs_ref[0:N, :] = h * dinv2
    hs_ref[N:N_P, :] = jnp.zeros((N_P - N, H), jnp.float32)


def _tc_final(acc_ref, dinv_ref, b3_ref, out_ref):
    a = acc_ref[0] + acc_ref[1]
    pre = a[0:N, 0:C] * dinv_ref[0:N] + b3_ref[...]
    m = jnp.max(pre, axis=1, keepdims=True)
    z = pre - m
    lse = jnp.log(jnp.sum(jnp.exp(z), axis=1, keepdims=True))
    out_ref[...] = z - lse


def kernel(x, adj_t, W1, b1, g1, be1, W2, b2, g2, be2, W3, b3):
    src = adj_t[0]
    dst = adj_t[1]
    pad = jnp.full((E_PAD - E,), N_P - 1, jnp.int32)
    src3 = jnp.concatenate([src, pad]).reshape(NW, NCHUNK, CHUNK)
    dst3 = jnp.concatenate([dst, pad]).reshape(NW, NCHUNK, CHUNK)
    zeros1 = jnp.zeros((ROWS_PER_TILE,), jnp.float32)
    zeros_h = jnp.zeros((ROWS_PER_TILE, H), jnp.float32)
    W3p = jnp.pad(W3, ((0, 0), (0, H - C)))

    deg2 = _sc_degree(dst3, zeros1)

    hs1, dinv = pl.pallas_call(
        _tc_first,
        out_shape=(jax.ShapeDtypeStruct((N_P, H), jnp.float32),
                   jax.ShapeDtypeStruct((N_P, 1), jnp.float32)),
    )(deg2, x, W1)

    acc1 = _sc_aggregate(hs1, src3, dst3, zeros_h)
    hs2 = pl.pallas_call(
        _tc_mid,
        out_shape=jax.ShapeDtypeStruct((N_P, H), jnp.float32),
    )(acc1, dinv, b1, g1, be1, W2)

    acc2 = _sc_aggregate(hs2, src3, dst3, zeros_h)
    hs3 = pl.pallas_call(
        _tc_mid,
        out_shape=jax.ShapeDtypeStruct((N_P, H), jnp.float32),
    )(acc2, dinv, b2, g2, be2, W3p)

    acc3 = _sc_aggregate(hs3, src3, dst3, zeros_h)
    out = pl.pallas_call(
        _tc_final,
        out_shape=jax.ShapeDtypeStruct((N, C), jnp.float32),
    )(acc3, dinv, b3)
    return out
